# baseline (speedup 1.0000x reference)
import functools

import jax
import jax.numpy as jnp
from jax import lax
from jax.experimental import pallas as pl
from jax.experimental.pallas import tpu as pltpu
from jax.experimental.pallas import tpu_sc as plsc

NUM_USERS = 5000
NUM_ITEMS = 5000
N = NUM_USERS + NUM_ITEMS
E = 320000
D = 128
DH = D // 2

NC = 2
NS = 16
EPT = E // NS
C = 80
NCHUNK = EPT // C
SEG = 25
NSEG = NCHUNK // SEG
NBUF = 3
RPS = N // NS


@functools.cache
def _make_sc_aggregate():
    mesh = plsc.VectorSubcoreMesh(core_axis_name="c", subcore_axis_name="s")

    @functools.partial(
        pl.kernel,
        mesh=mesh,
        compiler_params=pltpu.CompilerParams(use_tc_tiling_on_sc=False),
        out_type=jax.ShapeDtypeStruct((NC, NS, RPS, DH), jnp.float32),
        scratch_types=[
            pltpu.VMEM((2, SEG, C), jnp.int32),
            pltpu.VMEM((2, SEG, C), jnp.int32),
            pltpu.VMEM((NCHUNK, C), jnp.float32),
            pltpu.VMEM((NBUF, C, DH), jnp.float32),
            pltpu.VMEM_SHARED((N, DH), jnp.float32),
            pltpu.SemaphoreType.DMA,
            pltpu.SemaphoreType.DMA,
            pltpu.SemaphoreType.DMA,
            pltpu.SemaphoreType.DMA,
            pltpu.SemaphoreType.DMA,
            pltpu.SemaphoreType.DMA,
            pltpu.SemaphoreType.DMA,
        ],
    )
    def _sc_aggregate(ego_hbm, src_hbm, dst_hbm, w_hbm, out_hbm,
                      src_v, dst_v, w_v, rows_v, acc_sh,
                      sem_g0, sem_g1, sem_g2, sem_s0, sem_s1, sem_s2, sem_i):
        c = lax.axis_index("c")
        s = lax.axis_index("s")
        sems_g = [sem_g0, sem_g1, sem_g2]
        sems_s = [sem_s0, sem_s1, sem_s2]

        cp_w = pltpu.async_copy(w_hbm.at[s], w_v, sem_i)

        zvec = jnp.zeros((16,), jnp.float32)

        def _zrow(i, carry):
            for k in range(DH // 16):
                rows_v[0, i, pl.ds(k * 16, 16)] = zvec
            return carry

        lax.fori_loop(0, C, _zrow, 0)
        base = s * RPS
        for q in range(RPS // C):
            pltpu.sync_copy(rows_v.at[0], acc_sh.at[pl.ds(base + q * C, C)])
        rem = RPS % C
        if rem:
            pltpu.sync_copy(rows_v.at[0, pl.ds(0, rem)],
                            acc_sh.at[pl.ds(base + (RPS // C) * C, rem)])
        cp_w.wait()

        pltpu.sync_copy(src_hbm.at[c, s, 0], src_v.at[0])
        pltpu.sync_copy(dst_hbm.at[s, 0], dst_v.at[0])

        plsc.subcore_barrier()

        pltpu.async_copy(ego_hbm.at[src_v.at[0, 0]], rows_v.at[0], sem_g0)

        def _chunk(j, carry):
            jn = j + 1
            sg = lax.div(j, SEG)
            sgn = lax.div(jn, SEG)
            jn_in_seg = lax.rem(jn, SEG)

            @pl.when(jnp.logical_and(jn < NCHUNK, jn_in_seg == 0))
            def _drain_idx():
                pltpu.make_async_copy(
                    src_hbm.at[0, 0, 0], src_v.at[lax.rem(sgn, 2)],
                    sem_i).wait()
                pltpu.make_async_copy(
                    dst_hbm.at[0, 0], dst_v.at[lax.rem(sgn, 2)],
                    sem_i).wait()

            for bb in range(NBUF):
                @pl.when(jnp.logical_and(jn < NCHUNK, lax.rem(jn, NBUF) == bb))
                def _prefetch(bb=bb):
                    @pl.when(j >= NBUF - 1)
                    def _drain_scatter():
                        pltpu.make_async_copy(
                            ego_hbm.at[pl.ds(0, C)], rows_v.at[bb],
                            sems_s[bb]).wait()

                    pltpu.async_copy(
                        ego_hbm.at[src_v.at[lax.rem(sgn, 2), jn_in_seg]],
                        rows_v.at[bb], sems_g[bb])

            def _scale_buf(bb):
                def _scale(g, cc):
                    wv = w_v[j, pl.ds(g * 16, 16)]
                    for l in range(16):
                        wi = wv[l]
                        i = g * 16 + l
                        for k in range(DH // 16):
                            sl = pl.ds(k * 16, 16)
                            rows_v[bb, i, sl] = rows_v[bb, i, sl] * wi
                    return cc

                lax.fori_loop(0, C // 16, _scale, 0)

            for bb in range(NBUF):
                @pl.when(lax.rem(j, NBUF) == bb)
                def _process(bb=bb):
                    pltpu.make_async_copy(
                        ego_hbm.at[pl.ds(0, C)], rows_v.at[bb],
                        sems_g[bb]).wait()
                    _scale_buf(bb)
                    pltpu.async_copy(
                        rows_v.at[bb],
                        acc_sh.at[dst_v.at[lax.rem(sg, 2), lax.rem(j, SEG)]],
                        sems_s[bb], add=True)

            @pl.when(lax.rem(j, SEG) == NBUF - 1)
            def _issue_idx():
                @pl.when(sg + 1 < NSEG)
                def _issue():
                    pltpu.async_copy(
                        src_hbm.at[c, s, sg + 1],
                        src_v.at[lax.rem(sg + 1, 2)], sem_i)
                    pltpu.async_copy(
                        dst_hbm.at[s, sg + 1],
                        dst_v.at[lax.rem(sg + 1, 2)], sem_i)

            return carry

        lax.fori_loop(0, NCHUNK, _chunk, 0)

        for bb in range(NBUF):
            pltpu.make_async_copy(
                ego_hbm.at[pl.ds(0, C)], rows_v.at[bb], sems_s[bb]).wait()

        plsc.subcore_barrier()
        pltpu.sync_copy(acc_sh.at[pl.ds(s * RPS, RPS)], out_hbm.at[c, s])

    return _sc_aggregate


def _tc_layer_body(p_ref, gw_ref, gb_ref, bw_ref, bb_ref, ego_ref, mlp_ref):
    side = jnp.concatenate([p_ref[0], p_ref[1]], axis=1)
    h = lax.dot_general(side, gw_ref[...], (((1,), (1,)), ((), ())),
                        preferred_element_type=jnp.float32) + gb_ref[...]
    h = jnp.where(h >= 0, h, 0.01 * h)
    ego_ref[0] = h[:, :DH]
    ego_ref[1] = h[:, DH:]
    mlp_ref[...] = lax.dot_general(h, bw_ref[...], (((1,), (1,)), ((), ())),
                                   preferred_element_type=jnp.float32) + bb_ref[...]


ROWS_BLK = 1000


def _tc_layer(p, gw, gb, bw, bb):
    return pl.pallas_call(
        _tc_layer_body,
        grid=(N // ROWS_BLK,),
        in_specs=[
            pl.BlockSpec((NC, ROWS_BLK, DH), lambda i: (0, i, 0)),
            pl.BlockSpec((D, D), lambda i: (0, 0)),
            pl.BlockSpec((1, D), lambda i: (0, 0)),
            pl.BlockSpec((D, D), lambda i: (0, 0)),
            pl.BlockSpec((1, D), lambda i: (0, 0)),
        ],
        out_specs=[
            pl.BlockSpec((NC, ROWS_BLK, DH), lambda i: (0, i, 0)),
            pl.BlockSpec((ROWS_BLK, D), lambda i: (i, 0)),
        ],
        out_shape=[
            jax.ShapeDtypeStruct((NC, N, DH), jnp.float32),
            jax.ShapeDtypeStruct((N, D), jnp.float32),
        ],
    )(p, gw, gb, bw, bb)


def kernel(edge_index, edge_weight, emb_user, emb_item,
           gc_W0, gc_b0, gc_W1, gc_b1, bi_W0, bi_b0, bi_W1, bi_b1):
    src0 = edge_index[0].reshape(1, NS, NSEG, SEG, C)
    src = jnp.concatenate([src0, src0 + N], axis=0)
    dst = edge_index[1].reshape(NS, NSEG, SEG, C)
    w = edge_weight.reshape(NS, NCHUNK, C)

    ego0 = jnp.concatenate([emb_user, emb_item], axis=0)
    ego0_halves = jnp.stack([ego0[:, :DH], ego0[:, DH:]])

    sc_aggregate = _make_sc_aggregate()
    tcl = _tc_layer

    p0 = sc_aggregate(ego0_halves.reshape(NC * N, DH), src, dst, w)
    ego1_h, mlp0 = tcl(p0.reshape(NC, N, DH), gc_W0, gc_b0.reshape(1, D),
                       bi_W0, bi_b0.reshape(1, D))
    p1 = sc_aggregate(ego1_h.reshape(NC * N, DH), src, dst, w)
    _, mlp1 = tcl(p1.reshape(NC, N, DH), gc_W1, gc_b1.reshape(1, D),
                  bi_W1, bi_b1.reshape(1, D))

    users = jnp.concatenate(
        [ego0[:NUM_USERS], mlp0[:NUM_USERS], mlp1[:NUM_USERS]], axis=1)
    items = jnp.concatenate(
        [ego0[NUM_USERS:], mlp0[NUM_USERS:], mlp1[NUM_USERS:]], axis=1)
    return (users, items)


# speedup vs baseline: 2.1571x; 2.1571x over previous
import functools

import jax
import jax.numpy as jnp
from jax import lax
from jax.experimental import pallas as pl
from jax.experimental.pallas import tpu as pltpu
from jax.experimental.pallas import tpu_sc as plsc

NUM_USERS = 5000
NUM_ITEMS = 5000
N = NUM_USERS + NUM_ITEMS
E = 320000
D = 128

NC = 2
NS = 16
NW = NC * NS
EPW = E // NW
C = 80
NCHUNK = EPW // C
SEG = 5
NSEG = NCHUNK // SEG
NBUF = 2
RPS = N // NS


@functools.cache
def _make_sc_aggregate():
    mesh = plsc.VectorSubcoreMesh(core_axis_name="c", subcore_axis_name="s")

    @functools.partial(
        pl.kernel,
        mesh=mesh,
        out_type=jax.ShapeDtypeStruct((NC, NS, RPS, D), jnp.float32),
        scratch_types=[
            pltpu.VMEM((2, SEG, C), jnp.int32),
            pltpu.VMEM((2, SEG, C), jnp.int32),
            pltpu.VMEM((2, SEG, C), jnp.float32),
            pltpu.VMEM((NBUF, C, D), jnp.float32),
            pltpu.VMEM_SHARED((N, D), jnp.float32),
            pltpu.SemaphoreType.DMA,
            pltpu.SemaphoreType.DMA,
            pltpu.SemaphoreType.DMA,
            pltpu.SemaphoreType.DMA,
            pltpu.SemaphoreType.DMA,
        ],
    )
    def _sc_aggregate(ego_hbm, src_hbm, dst_hbm, w_hbm, out_hbm,
                      src_v, dst_v, w_v, rows_v, acc_sh,
                      sem_g0, sem_g1, sem_s0, sem_s1, sem_i):
        c = lax.axis_index("c")
        s = lax.axis_index("s")
        wid = c * NS + s
        sems_g = [sem_g0, sem_g1]
        sems_s = [sem_s0, sem_s1]

        zvec = jnp.zeros((16,), jnp.float32)

        def _zrow(i, carry):
            for k in range(D // 16):
                rows_v[0, i, pl.ds(k * 16, 16)] = zvec
            return carry

        lax.fori_loop(0, C, _zrow, 0)
        base = s * RPS
        for q in range(RPS // C):
            pltpu.sync_copy(rows_v.at[0], acc_sh.at[pl.ds(base + q * C, C)])
        rem = RPS % C
        if rem:
            pltpu.sync_copy(rows_v.at[0, pl.ds(0, rem)],
                            acc_sh.at[pl.ds(base + (RPS // C) * C, rem)])

        pltpu.sync_copy(src_hbm.at[wid, 0], src_v.at[0])
        pltpu.sync_copy(dst_hbm.at[wid, 0], dst_v.at[0])
        pltpu.sync_copy(w_hbm.at[wid, 0], w_v.at[0])

        plsc.subcore_barrier()

        pltpu.async_copy(ego_hbm.at[src_v.at[0, 0]], rows_v.at[0], sem_g0)

        def _chunk(j, carry):
            jn = j + 1
            sg = lax.div(j, SEG)
            sgn = lax.div(jn, SEG)
            jn_in_seg = lax.rem(jn, SEG)

            @pl.when(jnp.logical_and(jn < NCHUNK, jn_in_seg == 0))
            def _drain_idx():
                pltpu.make_async_copy(
                    src_hbm.at[0, 0], src_v.at[lax.rem(sgn, 2)],
                    sem_i).wait()
                pltpu.make_async_copy(
                    dst_hbm.at[0, 0], dst_v.at[lax.rem(sgn, 2)],
                    sem_i).wait()
                pltpu.make_async_copy(
                    w_hbm.at[0, 0], w_v.at[lax.rem(sgn, 2)],
                    sem_i).wait()

            for bb in range(NBUF):
                @pl.when(jnp.logical_and(jn < NCHUNK,
                                         lax.rem(jn, NBUF) == bb))
                def _prefetch(bb=bb):
                    @pl.when(j >= NBUF - 1)
                    def _drain_scatter():
                        pltpu.make_async_copy(
                            ego_hbm.at[pl.ds(0, C)], rows_v.at[bb],
                            sems_s[bb]).wait()

                    pltpu.async_copy(
                        ego_hbm.at[src_v.at[lax.rem(sgn, 2), jn_in_seg]],
                        rows_v.at[bb], sems_g[bb])

            def _scale_buf(bb, ip):
                def _scale(g, cc):
                    wv = w_v[ip, lax.rem(j, SEG), pl.ds(g * 16, 16)]
                    for l in range(16):
                        wi = wv[l]
                        i = g * 16 + l
                        for k in range(D // 16):
                            sl = pl.ds(k * 16, 16)
                            rows_v[bb, i, sl] = rows_v[bb, i, sl] * wi
                    return cc

                lax.fori_loop(0, C // 16, _scale, 0)

            for bb in range(NBUF):
                @pl.when(lax.rem(j, NBUF) == bb)
                def _process(bb=bb):
                    pltpu.make_async_copy(
                        ego_hbm.at[pl.ds(0, C)], rows_v.at[bb],
                        sems_g[bb]).wait()
                    for ip in range(2):
                        @pl.when(lax.rem(sg, 2) == ip)
                        def _scale_ip(bb=bb, ip=ip):
                            _scale_buf(bb, ip)
                    pltpu.async_copy(
                        rows_v.at[bb],
                        acc_sh.at[dst_v.at[lax.rem(sg, 2), lax.rem(j, SEG)]],
                        sems_s[bb], add=True)

            @pl.when(lax.rem(j, SEG) == NBUF - 1)
            def _issue_idx():
                @pl.when(sg + 1 < NSEG)
                def _issue():
                    pltpu.async_copy(
                        src_hbm.at[wid, sg + 1],
                        src_v.at[lax.rem(sg + 1, 2)], sem_i)
                    pltpu.async_copy(
                        dst_hbm.at[wid, sg + 1],
                        dst_v.at[lax.rem(sg + 1, 2)], sem_i)
                    pltpu.async_copy(
                        w_hbm.at[wid, sg + 1],
                        w_v.at[lax.rem(sg + 1, 2)], sem_i)

            return carry

        lax.fori_loop(0, NCHUNK, _chunk, 0)

        for bb in range(NBUF):
            pltpu.make_async_copy(
                ego_hbm.at[pl.ds(0, C)], rows_v.at[bb], sems_s[bb]).wait()

        plsc.subcore_barrier()
        pltpu.sync_copy(acc_sh.at[pl.ds(s * RPS, RPS)], out_hbm.at[c, s])

    return _sc_aggregate


def _tc_layer_body(p_ref, gw_ref, gb_ref, bw_ref, bb_ref, ego_ref, mlp_ref):
    side = p_ref[0] + p_ref[1]
    h = lax.dot_general(side, gw_ref[...], (((1,), (1,)), ((), ())),
                        preferred_element_type=jnp.float32) + gb_ref[...]
    h = jnp.where(h >= 0, h, 0.01 * h)
    ego_ref[...] = h
    mlp_ref[...] = lax.dot_general(h, bw_ref[...], (((1,), (1,)), ((), ())),
                                   preferred_element_type=jnp.float32) + bb_ref[...]


ROWS_BLK = 1000


def _tc_layer(p, gw, gb, bw, bb):
    return pl.pallas_call(
        _tc_layer_body,
        grid=(N // ROWS_BLK,),
        in_specs=[
            pl.BlockSpec((NC, ROWS_BLK, D), lambda i: (0, i, 0)),
            pl.BlockSpec((D, D), lambda i: (0, 0)),
            pl.BlockSpec((1, D), lambda i: (0, 0)),
            pl.BlockSpec((D, D), lambda i: (0, 0)),
            pl.BlockSpec((1, D), lambda i: (0, 0)),
        ],
        out_specs=[
            pl.BlockSpec((ROWS_BLK, D), lambda i: (i, 0)),
            pl.BlockSpec((ROWS_BLK, D), lambda i: (i, 0)),
        ],
        out_shape=[
            jax.ShapeDtypeStruct((N, D), jnp.float32),
            jax.ShapeDtypeStruct((N, D), jnp.float32),
        ],
    )(p, gw, gb, bw, bb)


def kernel(edge_index, edge_weight, emb_user, emb_item,
           gc_W0, gc_b0, gc_W1, gc_b1, bi_W0, bi_b0, bi_W1, bi_b1):
    src = edge_index[0].reshape(NW, NSEG, SEG, C)
    dst = edge_index[1].reshape(NW, NSEG, SEG, C)
    w = edge_weight.reshape(NW, NSEG, SEG, C)
    ego0 = jnp.concatenate([emb_user, emb_item], axis=0)

    sc_aggregate = _make_sc_aggregate()
    p0 = sc_aggregate(ego0, src, dst, w).reshape(NC, N, D)
    ego1, mlp0 = _tc_layer(p0, gc_W0, gc_b0.reshape(1, D),
                           bi_W0, bi_b0.reshape(1, D))
    p1 = sc_aggregate(ego1, src, dst, w).reshape(NC, N, D)
    _, mlp1 = _tc_layer(p1, gc_W1, gc_b1.reshape(1, D),
                        bi_W1, bi_b1.reshape(1, D))

    users = jnp.concatenate(
        [ego0[:NUM_USERS], mlp0[:NUM_USERS], mlp1[:NUM_USERS]], axis=1)
    items = jnp.concatenate(
        [ego0[NUM_USERS:], mlp0[NUM_USERS:], mlp1[NUM_USERS:]], axis=1)
    return (users, items)
